# baseline (speedup 1.0000x reference)
import functools

import jax
import jax.numpy as jnp
from jax import lax
from jax.experimental import pallas as pl
from jax.experimental.pallas import tpu as pltpu
from jax.experimental.pallas import tpu_sc as plsc

N = 10000
NP = 10240
E = 320000
EP = 327680
F0, F1, F2, F3 = 128, 2048, 256, 128
NTILES = 32
EPT = EP // NTILES
CH = 128
NCH = EPT // CH
SPT = NP // 16

_MESH = plsc.VectorSubcoreMesh(core_axis_name="c", subcore_axis_name="s")
_SC_PARAMS = pltpu.CompilerParams(needs_layout_passes=False)


@functools.partial(
    pl.kernel,
    out_type=jax.ShapeDtypeStruct((2, NP), jnp.float32),
    mesh=_MESH,
    compiler_params=_SC_PARAMS,
    scratch_types=[
        pltpu.VMEM((NCH, CH), jnp.int32),
        pltpu.VMEM((EPT,), jnp.float32),
        pltpu.VMEM((SPT,), jnp.float32),
        pltpu.VMEM_SHARED((NP,), jnp.float32),
    ],
)
def _deg(dst_h, w_h, out_h, dstb, wb, stgb, shared):
    c = lax.axis_index("c")
    s = lax.axis_index("s")
    b = c * 16 + s
    zero16 = jnp.zeros((16,), jnp.float32)

    def zr(r, carry):
        stgb[pl.ds(r * 16, 16)] = zero16
        return carry

    lax.fori_loop(0, SPT // 16, zr, 0)
    pltpu.sync_copy(stgb, shared.at[pl.ds(s * SPT, SPT)])
    plsc.subcore_barrier()

    pltpu.sync_copy(dst_h.at[b], dstb)
    pltpu.sync_copy(w_h.at[b], wb)

    def chunk(j, carry):
        pltpu.sync_copy(wb.at[pl.ds(j * CH, CH)], shared.at[dstb.at[j]],
                        add=True)
        return carry

    lax.fori_loop(0, NCH, chunk, 0)
    plsc.subcore_barrier()

    pltpu.sync_copy(shared.at[pl.ds(s * SPT, SPT)], stgb)
    pltpu.sync_copy(stgb, out_h.at[c, pl.ds(s * SPT, SPT)])


@functools.partial(
    pl.kernel,
    out_type=jax.ShapeDtypeStruct((NTILES, EPT), jnp.float32),
    mesh=_MESH,
    compiler_params=_SC_PARAMS,
    scratch_types=[
        pltpu.VMEM((EPT,), jnp.int32),
        pltpu.VMEM((EPT,), jnp.float32),
        pltpu.VMEM((NP,), jnp.float32),
    ],
)
def _escale(src_h, w_h, dis_h, out_h, srcb, wb, disv):
    c = lax.axis_index("c")
    s = lax.axis_index("s")
    b = c * 16 + s
    pltpu.sync_copy(src_h.at[b], srcb)
    pltpu.sync_copy(w_h.at[b], wb)
    pltpu.sync_copy(dis_h, disv)

    def g(i, carry):
        sl = pl.ds(i * 16, 16)
        wb[sl] = wb[sl] * plsc.load_gather(disv, [srcb[sl]])
        return carry

    lax.fori_loop(0, EPT // 16, g, 0)
    pltpu.sync_copy(wb, out_h.at[b])


@functools.partial(
    pl.kernel,
    out_type=jax.ShapeDtypeStruct((2, NP, 128), jnp.float32),
    mesh=_MESH,
    compiler_params=_SC_PARAMS,
    scratch_types=[
        pltpu.VMEM((2, 2, CH), jnp.int32),
        pltpu.VMEM((2 * CH,), jnp.float32),
        pltpu.VMEM((2, CH, 128), jnp.float32),
        pltpu.SemaphoreType.DMA((2,)),
        pltpu.SemaphoreType.DMA((2,)),
        pltpu.VMEM_SHARED((NP, 128), jnp.float32),
    ],
)
def _agg(m_h, e_h, sc_h, out_h, idxb, scb, rowb, gsem, ssem, shared):
    c = lax.axis_index("c")
    s = lax.axis_index("s")
    b = c * 16 + s
    zero16 = jnp.zeros((16,), jnp.float32)

    def zr(r, carry):
        for k in range(8):
            rowb[0, r, pl.ds(k * 16, 16)] = zero16
        return carry

    lax.fori_loop(0, CH, zr, 0)
    for m in range(SPT // CH):
        pltpu.sync_copy(rowb.at[0], shared.at[pl.ds(s * SPT + m * CH, CH), :])
    plsc.subcore_barrier()

    def load_chunk(j, p):
        pltpu.sync_copy(e_h.at[b, j], idxb.at[p])
        pltpu.sync_copy(sc_h.at[b, pl.ds(j * CH, CH)],
                        scb.at[pl.ds(p * CH, CH)])

    load_chunk(0, 0)
    pltpu.async_copy(m_h.at[idxb.at[0, 0]], rowb.at[0], gsem.at[0])

    def chunk(j, carry):
        p = jnp.bitwise_and(j, 1)
        q = 1 - p

        @pl.when(j + 1 < NCH)
        def _():
            @pl.when(j > 0)
            def _():
                pltpu.make_async_copy(rowb.at[q], shared.at[idxb.at[q, 1]],
                                      ssem.at[q]).wait()

            load_chunk(j + 1, q)
            pltpu.async_copy(m_h.at[idxb.at[q, 0]], rowb.at[q], gsem.at[q])

        pltpu.make_async_copy(m_h.at[idxb.at[p, 0]], rowb.at[p],
                              gsem.at[p]).wait()
        base = p * CH
        for e in range(CH):
            sp = plsc.load_gather(scb, [jnp.broadcast_to(base + e, (16,))])
            for k in range(8):
                sl = pl.ds(k * 16, 16)
                rowb[p, e, sl] = rowb[p, e, sl] * sp
        pltpu.async_copy(rowb.at[p], shared.at[idxb.at[p, 1]], ssem.at[p],
                         add=True)
        return carry

    lax.fori_loop(0, NCH, chunk, 0)

    def drain(pp, carry):
        pltpu.make_async_copy(rowb.at[pp], shared.at[idxb.at[pp, 1]],
                              ssem.at[pp]).wait()
        return carry

    lax.fori_loop(0, 2, drain, 0)
    plsc.subcore_barrier()

    for m in range(SPT // CH):
        pltpu.sync_copy(shared.at[pl.ds(s * SPT + m * CH, CH), :], rowb.at[0])
        pltpu.sync_copy(rowb.at[0], out_h.at[c, pl.ds(s * SPT + m * CH, CH), :])


def _prep(d0, d1):
    bm = 1024

    def body(d0_ref, d1_ref, dis_ref, disb_ref):
        deg = d0_ref[...] + d1_ref[...] + 1.0
        dis = lax.rsqrt(deg)
        dis_ref[...] = dis
        disb_ref[...] = jnp.broadcast_to(dis[:, None], (bm, 128))

    return pl.pallas_call(
        body,
        grid=(NP // bm,),
        in_specs=[
            pl.BlockSpec((bm,), lambda i: (i,)),
            pl.BlockSpec((bm,), lambda i: (i,)),
        ],
        out_specs=[
            pl.BlockSpec((bm,), lambda i: (i,)),
            pl.BlockSpec((bm, 128), lambda i: (i, 0)),
        ],
        out_shape=[
            jax.ShapeDtypeStruct((NP,), jnp.float32),
            jax.ShapeDtypeStruct((NP, 128), jnp.float32),
        ],
    )(d0, d1)


def _m1(p, x, dis_b, W1, b1):
    bm, bn = 1024, 512

    def body(p_ref, x_ref, d_ref, w_ref, b_ref, o_ref):
        d = d_ref[...]
        a = d * (p_ref[0] + p_ref[1] + d * x_ref[...])
        h = jnp.dot(a, w_ref[...], preferred_element_type=jnp.float32)
        h = h + b_ref[...][None, :]
        o_ref[...] = jnp.where(h >= 0, h, 0.01 * h)

    return pl.pallas_call(
        body,
        grid=(NP // bm, F1 // bn),
        in_specs=[
            pl.BlockSpec((2, bm, 128), lambda i, j: (0, i, 0)),
            pl.BlockSpec((bm, 128), lambda i, j: (i, 0)),
            pl.BlockSpec((bm, 128), lambda i, j: (i, 0)),
            pl.BlockSpec((128, bn), lambda i, j: (0, j)),
            pl.BlockSpec((bn,), lambda i, j: (j,)),
        ],
        out_specs=pl.BlockSpec((bm, bn), lambda i, j: (i, j)),
        out_shape=jax.ShapeDtypeStruct((NP, F1), jnp.float32),
    )(p, x, dis_b, W1, b1)


def _m2(h1, W2):
    bm = 1024

    def body(h_ref, w_ref, oa_ref, ob_ref):
        t = jnp.dot(h_ref[...], w_ref[...], preferred_element_type=jnp.float32)
        oa_ref[...] = t[:, :128]
        ob_ref[...] = t[:, 128:]

    return pl.pallas_call(
        body,
        grid=(NP // bm,),
        in_specs=[
            pl.BlockSpec((bm, F1), lambda i: (i, 0)),
            pl.BlockSpec((F1, F2), lambda i: (0, 0)),
        ],
        out_specs=[
            pl.BlockSpec((bm, 128), lambda i: (i, 0)),
            pl.BlockSpec((bm, 128), lambda i: (i, 0)),
        ],
        out_shape=[
            jax.ShapeDtypeStruct((NP, 128), jnp.float32),
            jax.ShapeDtypeStruct((NP, 128), jnp.float32),
        ],
    )(h1, W2)


def _m3(qa, qb, t2a, t2b, dis_b, b2, W3):
    bm = 1024

    def body(qa_ref, qb_ref, ta_ref, tb_ref, d_ref, b2_ref, w3_ref, o_ref):
        d = d_ref[...]
        b2v = b2_ref[...]
        h2a = d * (qa_ref[0] + qa_ref[1] + d * ta_ref[...]) + b2v[None, :128]
        h2b = d * (qb_ref[0] + qb_ref[1] + d * tb_ref[...]) + b2v[None, 128:]
        h2a = jnp.maximum(h2a, 0.0)
        h2b = jnp.maximum(h2b, 0.0)
        w3 = w3_ref[...]
        o_ref[...] = (
            jnp.dot(h2a, w3[:128], preferred_element_type=jnp.float32)
            + jnp.dot(h2b, w3[128:], preferred_element_type=jnp.float32))

    return pl.pallas_call(
        body,
        grid=(NP // bm,),
        in_specs=[
            pl.BlockSpec((2, bm, 128), lambda i: (0, i, 0)),
            pl.BlockSpec((2, bm, 128), lambda i: (0, i, 0)),
            pl.BlockSpec((bm, 128), lambda i: (i, 0)),
            pl.BlockSpec((bm, 128), lambda i: (i, 0)),
            pl.BlockSpec((bm, 128), lambda i: (i, 0)),
            pl.BlockSpec((F2,), lambda i: (0,)),
            pl.BlockSpec((F2, F3), lambda i: (0, 0)),
        ],
        out_specs=pl.BlockSpec((bm, 128), lambda i: (i, 0)),
        out_shape=jax.ShapeDtypeStruct((NP, F3), jnp.float32),
    )(qa, qb, t2a, t2b, dis_b, b2, W3)


def _final(r, t3, dis_b, b3):
    bm = 1024

    def body(r_ref, t_ref, d_ref, b_ref, o_ref):
        d = d_ref[...]
        h = d * (r_ref[0] + r_ref[1] + d * t_ref[...]) + b_ref[...][None, :]
        o_ref[...] = jnp.maximum(h, 0.0)

    return pl.pallas_call(
        body,
        grid=(NP // bm,),
        in_specs=[
            pl.BlockSpec((2, bm, 128), lambda i: (0, i, 0)),
            pl.BlockSpec((bm, 128), lambda i: (i, 0)),
            pl.BlockSpec((bm, 128), lambda i: (i, 0)),
            pl.BlockSpec((F3,), lambda i: (0,)),
        ],
        out_specs=pl.BlockSpec((bm, 128), lambda i: (i, 0)),
        out_shape=jax.ShapeDtypeStruct((NP, F3), jnp.float32),
    )(r, t3, dis_b, b3)


def kernel(x, edge_index, edge_weight, W1, b1, W2, b2, W3, b3):
    src = edge_index[0].astype(jnp.int32)
    dst = edge_index[1].astype(jnp.int32)
    w = edge_weight.astype(jnp.float32)

    src = jnp.pad(src, (0, EP - E))
    dst = jnp.pad(dst, (0, EP - E))
    w = jnp.pad(w, (0, EP - E))
    src3 = src.reshape(NTILES, NCH, CH)
    dst3 = dst.reshape(NTILES, NCH, CH)
    src2d = src.reshape(NTILES, EPT)
    w2d = w.reshape(NTILES, EPT)
    xp = jnp.pad(x, ((0, NP - N), (0, 0)))

    d = _deg(dst3, w2d)
    dis1, dis_b = _prep(d[0], d[1])
    sc = _escale(src2d, w2d, dis1)
    epk = jnp.stack([src3, dst3], axis=2)
    p = _agg(xp, epk, sc)
    h1 = _m1(p, xp, dis_b, W1, b1)
    t2a, t2b = _m2(h1, W2)
    qa = _agg(t2a, epk, sc)
    qb = _agg(t2b, epk, sc)
    t3 = _m3(qa, qb, t2a, t2b, dis_b, b2, W3)
    r = _agg(t3, epk, sc)
    out = _final(r, t3, dis_b, b3)
    return out[:N]


# speedup vs baseline: 1.0289x; 1.0289x over previous
import functools

import jax
import jax.numpy as jnp
from jax import lax
from jax.experimental import pallas as pl
from jax.experimental.pallas import tpu as pltpu
from jax.experimental.pallas import tpu_sc as plsc

N = 10000
NP = 10240
E = 320000
EP = 327680
F0, F1, F2, F3 = 128, 2048, 256, 128
NTILES = 32
EPT = EP // NTILES
CH = 128
NCH = EPT // CH
SPT = NP // 16

_MESH = plsc.VectorSubcoreMesh(core_axis_name="c", subcore_axis_name="s")
_SC_PARAMS = pltpu.CompilerParams(needs_layout_passes=False)


@functools.partial(
    pl.kernel,
    out_type=jax.ShapeDtypeStruct((2, NP), jnp.float32),
    mesh=_MESH,
    compiler_params=_SC_PARAMS,
    scratch_types=[
        pltpu.VMEM((NCH, CH), jnp.int32),
        pltpu.VMEM((EPT,), jnp.float32),
        pltpu.VMEM((SPT,), jnp.float32),
        pltpu.VMEM_SHARED((NP,), jnp.float32),
    ],
)
def _deg(dst_h, w_h, out_h, dstb, wb, stgb, shared):
    c = lax.axis_index("c")
    s = lax.axis_index("s")
    b = c * 16 + s
    zero16 = jnp.zeros((16,), jnp.float32)

    def zr(r, carry):
        stgb[pl.ds(r * 16, 16)] = zero16
        return carry

    lax.fori_loop(0, SPT // 16, zr, 0)
    pltpu.sync_copy(stgb, shared.at[pl.ds(s * SPT, SPT)])
    plsc.subcore_barrier()

    pltpu.sync_copy(dst_h.at[b], dstb)
    pltpu.sync_copy(w_h.at[b], wb)

    def chunk(j, carry):
        pltpu.sync_copy(wb.at[pl.ds(j * CH, CH)], shared.at[dstb.at[j]],
                        add=True)
        return carry

    lax.fori_loop(0, NCH, chunk, 0)
    plsc.subcore_barrier()

    pltpu.sync_copy(shared.at[pl.ds(s * SPT, SPT)], stgb)
    pltpu.sync_copy(stgb, out_h.at[c, pl.ds(s * SPT, SPT)])


@functools.partial(
    pl.kernel,
    out_type=jax.ShapeDtypeStruct((NTILES, EPT), jnp.float32),
    mesh=_MESH,
    compiler_params=_SC_PARAMS,
    scratch_types=[
        pltpu.VMEM((EPT,), jnp.int32),
        pltpu.VMEM((EPT,), jnp.float32),
        pltpu.VMEM((NP,), jnp.float32),
    ],
)
def _escale(src_h, w_h, dis_h, out_h, srcb, wb, disv):
    c = lax.axis_index("c")
    s = lax.axis_index("s")
    b = c * 16 + s
    pltpu.sync_copy(src_h.at[b], srcb)
    pltpu.sync_copy(w_h.at[b], wb)
    pltpu.sync_copy(dis_h, disv)

    def g(i, carry):
        sl = pl.ds(i * 16, 16)
        wb[sl] = wb[sl] * plsc.load_gather(disv, [srcb[sl]])
        return carry

    lax.fori_loop(0, EPT // 16, g, 0)
    pltpu.sync_copy(wb, out_h.at[b])


@functools.partial(
    pl.kernel,
    out_type=jax.ShapeDtypeStruct((2, NP, 128), jnp.float32),
    mesh=_MESH,
    compiler_params=_SC_PARAMS,
    scratch_types=[
        pltpu.VMEM((4, 2, CH), jnp.int32),
        pltpu.VMEM((4 * CH,), jnp.float32),
        pltpu.VMEM((2, CH, 128), jnp.float32),
        pltpu.SemaphoreType.DMA((2,)),
        pltpu.SemaphoreType.DMA((2,)),
        pltpu.SemaphoreType.DMA((4,)),
        pltpu.SemaphoreType.DMA((4,)),
        pltpu.VMEM_SHARED((NP, 128), jnp.float32),
    ],
)
def _agg(m_h, e_h, sc_h, out_h, idxb, scb, rowb, gsem, ssem, isem, csem,
         shared):
    c = lax.axis_index("c")
    s = lax.axis_index("s")
    b = c * 16 + s
    zero16 = jnp.zeros((16,), jnp.float32)

    def zr(r, carry):
        for k in range(8):
            rowb[0, r, pl.ds(k * 16, 16)] = zero16
        return carry

    lax.fori_loop(0, CH, zr, 0)
    for m in range(SPT // CH):
        pltpu.sync_copy(rowb.at[0], shared.at[pl.ds(s * SPT + m * CH, CH), :])
    plsc.subcore_barrier()

    def load_chunk(j, sl_):
        pltpu.async_copy(e_h.at[b, j], idxb.at[sl_], isem.at[sl_])
        pltpu.async_copy(sc_h.at[b, pl.ds(j * CH, CH)],
                         scb.at[pl.ds(sl_ * CH, CH)], csem.at[sl_])

    def wait_chunk(j, sl_):
        pltpu.make_async_copy(e_h.at[b, j], idxb.at[sl_],
                              isem.at[sl_]).wait()
        pltpu.make_async_copy(sc_h.at[b, pl.ds(j * CH, CH)],
                              scb.at[pl.ds(sl_ * CH, CH)],
                              csem.at[sl_]).wait()

    load_chunk(0, 0)
    load_chunk(1, 1)
    wait_chunk(0, 0)
    pltpu.async_copy(m_h.at[idxb.at[0, 0]], rowb.at[0], gsem.at[0])

    def chunk(j, carry):
        p = jnp.bitwise_and(j, 1)
        q = 1 - p
        sl_p = jnp.bitwise_and(j, 3)

        @pl.when(j + 2 < NCH)
        def _():
            load_chunk(j + 2, jnp.bitwise_and(j + 2, 3))

        @pl.when(j + 1 < NCH)
        def _():
            @pl.when(j > 0)
            def _():
                pltpu.make_async_copy(
                    rowb.at[q], shared.at[idxb.at[jnp.bitwise_and(j - 1, 3),
                                                  1]], ssem.at[q]).wait()

            sl_n = jnp.bitwise_and(j + 1, 3)
            wait_chunk(j + 1, sl_n)
            pltpu.async_copy(m_h.at[idxb.at[sl_n, 0]], rowb.at[q],
                             gsem.at[q])

        pltpu.make_async_copy(m_h.at[idxb.at[sl_p, 0]], rowb.at[p],
                              gsem.at[p]).wait()
        base = sl_p * CH
        for e in range(CH):
            sp = plsc.load_gather(scb, [jnp.broadcast_to(base + e, (16,))])
            for k in range(8):
                sl = pl.ds(k * 16, 16)
                rowb[p, e, sl] = rowb[p, e, sl] * sp
        pltpu.async_copy(rowb.at[p], shared.at[idxb.at[sl_p, 1]], ssem.at[p],
                         add=True)
        return carry

    lax.fori_loop(0, NCH, chunk, 0)

    def drain(pp, carry):
        pltpu.make_async_copy(rowb.at[pp], shared.at[idxb.at[pp, 1]],
                              ssem.at[pp]).wait()
        return carry

    lax.fori_loop(0, 2, drain, 0)
    plsc.subcore_barrier()

    for m in range(SPT // CH):
        pltpu.sync_copy(shared.at[pl.ds(s * SPT + m * CH, CH), :], rowb.at[0])
        pltpu.sync_copy(rowb.at[0], out_h.at[c, pl.ds(s * SPT + m * CH, CH), :])


def _prep(d0, d1):
    bm = 1024

    def body(d0_ref, d1_ref, dis_ref, disb_ref):
        deg = d0_ref[...] + d1_ref[...] + 1.0
        dis = lax.rsqrt(deg)
        dis_ref[...] = dis
        disb_ref[...] = jnp.broadcast_to(dis[:, None], (bm, 128))

    return pl.pallas_call(
        body,
        grid=(NP // bm,),
        in_specs=[
            pl.BlockSpec((bm,), lambda i: (i,)),
            pl.BlockSpec((bm,), lambda i: (i,)),
        ],
        out_specs=[
            pl.BlockSpec((bm,), lambda i: (i,)),
            pl.BlockSpec((bm, 128), lambda i: (i, 0)),
        ],
        out_shape=[
            jax.ShapeDtypeStruct((NP,), jnp.float32),
            jax.ShapeDtypeStruct((NP, 128), jnp.float32),
        ],
    )(d0, d1)


def _m1(p, x, dis_b, W1, b1):
    bm, bn = 1024, 512

    def body(p_ref, x_ref, d_ref, w_ref, b_ref, o_ref):
        d = d_ref[...]
        a = d * (p_ref[0] + p_ref[1] + d * x_ref[...])
        h = jnp.dot(a, w_ref[...], preferred_element_type=jnp.float32)
        h = h + b_ref[...][None, :]
        o_ref[...] = jnp.where(h >= 0, h, 0.01 * h)

    return pl.pallas_call(
        body,
        grid=(NP // bm, F1 // bn),
        in_specs=[
            pl.BlockSpec((2, bm, 128), lambda i, j: (0, i, 0)),
            pl.BlockSpec((bm, 128), lambda i, j: (i, 0)),
            pl.BlockSpec((bm, 128), lambda i, j: (i, 0)),
            pl.BlockSpec((128, bn), lambda i, j: (0, j)),
            pl.BlockSpec((bn,), lambda i, j: (j,)),
        ],
        out_specs=pl.BlockSpec((bm, bn), lambda i, j: (i, j)),
        out_shape=jax.ShapeDtypeStruct((NP, F1), jnp.float32),
    )(p, x, dis_b, W1, b1)


def _m2(h1, W2):
    bm = 1024

    def body(h_ref, w_ref, oa_ref, ob_ref):
        t = jnp.dot(h_ref[...], w_ref[...], preferred_element_type=jnp.float32)
        oa_ref[...] = t[:, :128]
        ob_ref[...] = t[:, 128:]

    return pl.pallas_call(
        body,
        grid=(NP // bm,),
        in_specs=[
            pl.BlockSpec((bm, F1), lambda i: (i, 0)),
            pl.BlockSpec((F1, F2), lambda i: (0, 0)),
        ],
        out_specs=[
            pl.BlockSpec((bm, 128), lambda i: (i, 0)),
            pl.BlockSpec((bm, 128), lambda i: (i, 0)),
        ],
        out_shape=[
            jax.ShapeDtypeStruct((NP, 128), jnp.float32),
            jax.ShapeDtypeStruct((NP, 128), jnp.float32),
        ],
    )(h1, W2)


def _m3(qa, qb, t2a, t2b, dis_b, b2, W3):
    bm = 1024

    def body(qa_ref, qb_ref, ta_ref, tb_ref, d_ref, b2_ref, w3_ref, o_ref):
        d = d_ref[...]
        b2v = b2_ref[...]
        h2a = d * (qa_ref[0] + qa_ref[1] + d * ta_ref[...]) + b2v[None, :128]
        h2b = d * (qb_ref[0] + qb_ref[1] + d * tb_ref[...]) + b2v[None, 128:]
        h2a = jnp.maximum(h2a, 0.0)
        h2b = jnp.maximum(h2b, 0.0)
        w3 = w3_ref[...]
        o_ref[...] = (
            jnp.dot(h2a, w3[:128], preferred_element_type=jnp.float32)
            + jnp.dot(h2b, w3[128:], preferred_element_type=jnp.float32))

    return pl.pallas_call(
        body,
        grid=(NP // bm,),
        in_specs=[
            pl.BlockSpec((2, bm, 128), lambda i: (0, i, 0)),
            pl.BlockSpec((2, bm, 128), lambda i: (0, i, 0)),
            pl.BlockSpec((bm, 128), lambda i: (i, 0)),
            pl.BlockSpec((bm, 128), lambda i: (i, 0)),
            pl.BlockSpec((bm, 128), lambda i: (i, 0)),
            pl.BlockSpec((F2,), lambda i: (0,)),
            pl.BlockSpec((F2, F3), lambda i: (0, 0)),
        ],
        out_specs=pl.BlockSpec((bm, 128), lambda i: (i, 0)),
        out_shape=jax.ShapeDtypeStruct((NP, F3), jnp.float32),
    )(qa, qb, t2a, t2b, dis_b, b2, W3)


def _final(r, t3, dis_b, b3):
    bm = 1024

    def body(r_ref, t_ref, d_ref, b_ref, o_ref):
        d = d_ref[...]
        h = d * (r_ref[0] + r_ref[1] + d * t_ref[...]) + b_ref[...][None, :]
        o_ref[...] = jnp.maximum(h, 0.0)

    return pl.pallas_call(
        body,
        grid=(NP // bm,),
        in_specs=[
            pl.BlockSpec((2, bm, 128), lambda i: (0, i, 0)),
            pl.BlockSpec((bm, 128), lambda i: (i, 0)),
            pl.BlockSpec((bm, 128), lambda i: (i, 0)),
            pl.BlockSpec((F3,), lambda i: (0,)),
        ],
        out_specs=pl.BlockSpec((bm, 128), lambda i: (i, 0)),
        out_shape=jax.ShapeDtypeStruct((NP, F3), jnp.float32),
    )(r, t3, dis_b, b3)


def kernel(x, edge_index, edge_weight, W1, b1, W2, b2, W3, b3):
    src = edge_index[0].astype(jnp.int32)
    dst = edge_index[1].astype(jnp.int32)
    w = edge_weight.astype(jnp.float32)

    src = jnp.pad(src, (0, EP - E))
    dst = jnp.pad(dst, (0, EP - E))
    w = jnp.pad(w, (0, EP - E))
    src3 = src.reshape(NTILES, NCH, CH)
    dst3 = dst.reshape(NTILES, NCH, CH)
    src2d = src.reshape(NTILES, EPT)
    w2d = w.reshape(NTILES, EPT)
    xp = jnp.pad(x, ((0, NP - N), (0, 0)))

    d = _deg(dst3, w2d)
    dis1, dis_b = _prep(d[0], d[1])
    sc = _escale(src2d, w2d, dis1)
    epk = jnp.stack([src3, dst3], axis=2)
    p = _agg(xp, epk, sc)
    h1 = _m1(p, xp, dis_b, W1, b1)
    t2a, t2b = _m2(h1, W2)
    qa = _agg(t2a, epk, sc)
    qb = _agg(t2b, epk, sc)
    t3 = _m3(qa, qb, t2a, t2b, dis_b, b2, W3)
    r = _agg(t3, epk, sc)
    out = _final(r, t3, dis_b, b3)
    return out[:N]
